# baseline (speedup 1.0000x reference)
import functools

import jax
import jax.numpy as jnp
from jax import lax
from jax.experimental import pallas as pl
from jax.experimental.pallas import tpu as pltpu
from jax.experimental.pallas import tpu_sc as plsc

HIDDEN_DIM = 512
BATCH = 16384
CHUNK = 64


def _make_kernel():
    info = plsc.get_sparse_core_info()
    nw = info.num_cores * info.num_subcores
    b_per_w = BATCH // nw
    n_chunks = b_per_w // CHUNK

    mesh = plsc.VectorSubcoreMesh(core_axis_name="c", subcore_axis_name="s")

    @functools.partial(
        pl.kernel,
        mesh=mesh,
        out_type=jax.ShapeDtypeStruct((BATCH, HIDDEN_DIM), jnp.float32),
        scratch_types=[
            pltpu.VMEM((b_per_w,), jnp.int32),
            pltpu.VMEM((CHUNK, HIDDEN_DIM), jnp.float32),
            pltpu.VMEM((CHUNK, HIDDEN_DIM), jnp.float32),
            pltpu.SemaphoreType.DMA,
            pltpu.SemaphoreType.DMA,
        ],
    )
    def k(table_hbm, idx_hbm, out_hbm, idx_v, buf0, buf1, sem_g, sem_s):
        wid = lax.axis_index("s") * info.num_cores + lax.axis_index("c")
        base = wid * b_per_w
        pltpu.sync_copy(idx_hbm.at[pl.ds(base, b_per_w)], idx_v)

        bufs = (buf0, buf1)
        gathers = [None] * n_chunks
        stores = [None] * n_chunks
        gathers[0] = pltpu.async_copy(
            table_hbm.at[idx_v.at[pl.ds(0, CHUNK)]], bufs[0], sem_g
        )
        for c in range(n_chunks):
            gathers[c].wait()
            if c + 1 < n_chunks:
                if c >= 1:
                    stores[c - 1].wait()
                gathers[c + 1] = pltpu.async_copy(
                    table_hbm.at[idx_v.at[pl.ds((c + 1) * CHUNK, CHUNK)]],
                    bufs[(c + 1) % 2],
                    sem_g,
                )
            stores[c] = pltpu.async_copy(
                bufs[c % 2], out_hbm.at[pl.ds(base + c * CHUNK, CHUNK)], sem_s
            )
        stores[n_chunks - 2].wait()
        stores[n_chunks - 1].wait()

    return k


_lookup = _make_kernel()


def kernel(domain_ids, embed_weight):
    return _lookup(embed_weight, domain_ids.astype(jnp.int32))


# speedup vs baseline: 4.8924x; 4.8924x over previous
import functools

import jax
import jax.numpy as jnp
from jax import lax
from jax.experimental import pallas as pl
from jax.experimental.pallas import tpu as pltpu
from jax.experimental.pallas import tpu_sc as plsc

HIDDEN_DIM = 512
BATCH = 16384
CHUNK = 64


def _make_kernel():
    info = plsc.get_sparse_core_info()
    nw = info.num_cores * info.num_subcores
    b_per_w = BATCH // nw
    n_chunks = b_per_w // CHUNK

    mesh = plsc.VectorSubcoreMesh(core_axis_name="c", subcore_axis_name="s")

    @functools.partial(
        pl.kernel,
        mesh=mesh,
        out_type=(
            jax.ShapeDtypeStruct((BATCH, HIDDEN_DIM), jnp.float32),
            jax.ShapeDtypeStruct((nw * 2, HIDDEN_DIM), jnp.float32),
        ),
        scratch_types=[
            pltpu.VMEM((2, HIDDEN_DIM), jnp.float32),
            pltpu.VMEM((b_per_w,), jnp.int32),
            pltpu.VMEM((CHUNK, HIDDEN_DIM), jnp.float32),
            pltpu.VMEM((CHUNK, HIDDEN_DIM), jnp.float32),
            pltpu.SemaphoreType.DMA,
            pltpu.SemaphoreType.DMA,
        ],
    )
    def k(table_hbm, idx_hbm, out_hbm, rep_hbm, table_v, idx_v, buf0, buf1,
          sem_g, sem_s):
        wid = lax.axis_index("s") * info.num_cores + lax.axis_index("c")
        base = wid * b_per_w
        pltpu.sync_copy(table_hbm, table_v)
        pltpu.sync_copy(idx_hbm.at[pl.ds(base, b_per_w)], idx_v)
        pltpu.sync_copy(table_v, rep_hbm.at[pl.ds(wid * 2, 2)])
        rebase = wid * 2
        for v in range(b_per_w // 16):
            sl = pl.ds(v * 16, 16)
            idx_v[sl] = idx_v[sl] + rebase

        bufs = (buf0, buf1)
        gathers = [None] * n_chunks
        stores = [None] * n_chunks
        gathers[0] = pltpu.async_copy(
            rep_hbm.at[idx_v.at[pl.ds(0, CHUNK)]], bufs[0], sem_g
        )
        for c in range(n_chunks):
            gathers[c].wait()
            if c + 1 < n_chunks:
                if c >= 1:
                    stores[c - 1].wait()
                gathers[c + 1] = pltpu.async_copy(
                    rep_hbm.at[idx_v.at[pl.ds((c + 1) * CHUNK, CHUNK)]],
                    bufs[(c + 1) % 2],
                    sem_g,
                )
            stores[c] = pltpu.async_copy(
                bufs[c % 2], out_hbm.at[pl.ds(base + c * CHUNK, CHUNK)], sem_s
            )
        stores[n_chunks - 2].wait()
        stores[n_chunks - 1].wait()

    return k


_lookup = _make_kernel()


def kernel(domain_ids, embed_weight):
    out, _ = _lookup(embed_weight, domain_ids.astype(jnp.int32))
    return out


# speedup vs baseline: 6.4873x; 1.3260x over previous
import functools

import jax
import jax.numpy as jnp
from jax import lax
from jax.experimental import pallas as pl
from jax.experimental.pallas import tpu as pltpu
from jax.experimental.pallas import tpu_sc as plsc

HIDDEN_DIM = 512
BATCH = 16384
CHUNK = 64
NREP = 4


def _make_kernel():
    info = plsc.get_sparse_core_info()
    nw = info.num_cores * info.num_subcores
    b_per_w = BATCH // nw
    n_chunks = b_per_w // CHUNK

    mesh = plsc.VectorSubcoreMesh(core_axis_name="c", subcore_axis_name="s")

    @functools.partial(
        pl.kernel,
        mesh=mesh,
        out_type=(
            jax.ShapeDtypeStruct((BATCH, HIDDEN_DIM), jnp.float32),
            jax.ShapeDtypeStruct((NREP * nw * 2, HIDDEN_DIM), jnp.float32),
        ),
        scratch_types=[
            pltpu.VMEM((2, HIDDEN_DIM), jnp.float32),
            pltpu.VMEM((b_per_w,), jnp.int32),
            pltpu.VMEM((CHUNK, HIDDEN_DIM), jnp.float32),
            pltpu.VMEM((CHUNK, HIDDEN_DIM), jnp.float32),
            pltpu.SemaphoreType.DMA,
            pltpu.SemaphoreType.DMA,
        ],
    )
    def k(table_hbm, idx_hbm, out_hbm, rep_hbm, table_v, idx_v, buf0, buf1,
          sem_g, sem_s):
        wid = lax.axis_index("s") * info.num_cores + lax.axis_index("c")
        base = wid * b_per_w
        pltpu.sync_copy(table_hbm, table_v)
        pltpu.sync_copy(idx_hbm.at[pl.ds(base, b_per_w)], idx_v)
        for r in range(NREP):
            pltpu.sync_copy(table_v, rep_hbm.at[pl.ds((r * nw + wid) * 2, 2)])
        pattern = (lax.iota(jnp.int32, 16) % NREP) * (nw * 2) + wid * 2
        for v in range(b_per_w // 16):
            sl = pl.ds(v * 16, 16)
            idx_v[sl] = idx_v[sl] + pattern

        bufs = (buf0, buf1)
        gathers = [None] * n_chunks
        stores = [None] * n_chunks
        gathers[0] = pltpu.async_copy(
            rep_hbm.at[idx_v.at[pl.ds(0, CHUNK)]], bufs[0], sem_g
        )
        for c in range(n_chunks):
            gathers[c].wait()
            if c + 1 < n_chunks:
                if c >= 1:
                    stores[c - 1].wait()
                gathers[c + 1] = pltpu.async_copy(
                    rep_hbm.at[idx_v.at[pl.ds((c + 1) * CHUNK, CHUNK)]],
                    bufs[(c + 1) % 2],
                    sem_g,
                )
            stores[c] = pltpu.async_copy(
                bufs[c % 2], out_hbm.at[pl.ds(base + c * CHUNK, CHUNK)], sem_s
            )
        stores[n_chunks - 2].wait()
        stores[n_chunks - 1].wait()

    return k


_lookup = _make_kernel()


def kernel(domain_ids, embed_weight):
    out, _ = _lookup(embed_weight, domain_ids.astype(jnp.int32))
    return out


# speedup vs baseline: 7.7478x; 1.1943x over previous
import functools

import jax
import jax.numpy as jnp
from jax import lax
from jax.experimental import pallas as pl
from jax.experimental.pallas import tpu as pltpu
from jax.experimental.pallas import tpu_sc as plsc

HIDDEN_DIM = 512
BATCH = 16384
CHUNK = 64
NREP = 16


def _make_kernel():
    info = plsc.get_sparse_core_info()
    nw = info.num_cores * info.num_subcores
    b_per_w = BATCH // nw
    n_chunks = b_per_w // CHUNK

    mesh = plsc.VectorSubcoreMesh(core_axis_name="c", subcore_axis_name="s")

    @functools.partial(
        pl.kernel,
        mesh=mesh,
        out_type=(
            jax.ShapeDtypeStruct((BATCH, HIDDEN_DIM), jnp.float32),
            jax.ShapeDtypeStruct((NREP * nw * 2, HIDDEN_DIM), jnp.float32),
        ),
        scratch_types=[
            pltpu.VMEM((2, HIDDEN_DIM), jnp.float32),
            pltpu.VMEM((b_per_w,), jnp.int32),
            pltpu.VMEM((CHUNK, HIDDEN_DIM), jnp.float32),
            pltpu.VMEM((CHUNK, HIDDEN_DIM), jnp.float32),
            pltpu.SemaphoreType.DMA,
            pltpu.SemaphoreType.DMA,
        ],
    )
    def k(table_hbm, idx_hbm, out_hbm, rep_hbm, table_v, idx_v, buf0, buf1,
          sem_g, sem_s):
        wid = lax.axis_index("s") * info.num_cores + lax.axis_index("c")
        base = wid * b_per_w
        pltpu.sync_copy(table_hbm, table_v)
        pltpu.sync_copy(idx_hbm.at[pl.ds(base, b_per_w)], idx_v)
        for r in range(NREP):
            pltpu.sync_copy(table_v, rep_hbm.at[pl.ds((r * nw + wid) * 2, 2)])
        pattern = (lax.iota(jnp.int32, 16) % NREP) * (nw * 2) + wid * 2
        for v in range(b_per_w // 16):
            sl = pl.ds(v * 16, 16)
            idx_v[sl] = idx_v[sl] + pattern

        bufs = (buf0, buf1)
        gathers = [None] * n_chunks
        stores = [None] * n_chunks
        gathers[0] = pltpu.async_copy(
            rep_hbm.at[idx_v.at[pl.ds(0, CHUNK)]], bufs[0], sem_g
        )
        for c in range(n_chunks):
            gathers[c].wait()
            if c + 1 < n_chunks:
                if c >= 1:
                    stores[c - 1].wait()
                gathers[c + 1] = pltpu.async_copy(
                    rep_hbm.at[idx_v.at[pl.ds((c + 1) * CHUNK, CHUNK)]],
                    bufs[(c + 1) % 2],
                    sem_g,
                )
            stores[c] = pltpu.async_copy(
                bufs[c % 2], out_hbm.at[pl.ds(base + c * CHUNK, CHUNK)], sem_s
            )
        stores[n_chunks - 2].wait()
        stores[n_chunks - 1].wait()

    return k


_lookup = _make_kernel()


def kernel(domain_ids, embed_weight):
    out, _ = _lookup(embed_weight, domain_ids.astype(jnp.int32))
    return out
